# baseline (speedup 1.0000x reference)
import functools

import jax
import jax.numpy as jnp
from jax import lax
from jax.experimental import pallas as pl
from jax.experimental.pallas import tpu as pltpu
from jax.experimental.pallas import tpu_sc as plsc

_EPS = 1e-6
_PI = 3.14159265358979323846
_HALF_PI = _PI / 2.0
_ATAN_C = (
    0.9999966347006731,
    -0.3331830289944677,
    0.19813213509068275,
    -0.1324752277162814,
    0.07981120495618609,
    -0.03372593810415406,
    0.006842624897572022,
)

_C = 1600
_IW = 80
_IROWS = _C // _IW


def _rsqrt(x):
    i = lax.bitcast_convert_type(x, jnp.int32)
    i = jnp.int32(0x5F3759DF) - lax.shift_right_arithmetic(i, 1)
    y = lax.bitcast_convert_type(i, jnp.float32)
    xh = x * 0.5
    y = y * (1.5 - xh * y * y)
    y = y * (1.5 - xh * y * y)
    y = y * (1.5 - xh * y * y)
    return y


def _atan2_pos(n, w):
    aw = jnp.abs(w)
    mn = jnp.minimum(n, aw)
    mx = jnp.maximum(n, aw)
    z = mn / jnp.maximum(mx, 1e-35)
    z2 = z * z
    p = jnp.float32(_ATAN_C[-1])
    for c in _ATAN_C[-2::-1]:
        p = p * z2 + jnp.float32(c)
    p = p * z
    r = jnp.where(n > aw, _HALF_PI - p, p)
    return jnp.where(w >= 0.0, r, _PI - r)


def _se3_log_parts(tx, ty, tz, qx, qy, qz, qw):
    n2 = qx * qx + qy * qy + qz * qz
    w2 = qw * qw
    s2 = n2 + w2
    n = n2 * _rsqrt(jnp.maximum(n2, 1e-35))
    theta = 2.0 * _atan2_pos(n, qw)
    n_safe = jnp.where(n > _EPS, n, 1.0)
    w_safe = jnp.where(jnp.abs(qw) > _EPS, qw, 1.0)
    scale = jnp.where(n > _EPS, theta / n_safe, 2.0 / w_safe)
    px = scale * qx
    py = scale * qy
    pz = scale * qz
    th = jnp.abs(scale) * n
    th_safe = jnp.where(th > _EPS, th, 1.0)
    inv_s2 = 1.0 / s2
    cth = (w2 - n2) * inv_s2
    sth = 2.0 * n * qw * inv_s2
    coef = jnp.where(
        th > _EPS,
        1.0 / (th_safe * th_safe)
        - (1.0 + cth) / (2.0 * th_safe * sth),
        1.0 / 12.0,
    )
    pvx = py * tz - pz * ty
    pvy = pz * tx - px * tz
    pvz = px * ty - py * tx
    ppvx = py * pvz - pz * pvy
    ppvy = pz * pvx - px * pvz
    ppvz = px * pvy - py * pvx
    rx = tx - 0.5 * pvx + coef * ppvx
    ry = ty - 0.5 * pvy + coef * ppvy
    rz = tz - 0.5 * pvz + coef * ppvz
    return rx, ry, rz, px, py, pz


def _edge_error_log(rel, n1, n2c):
    rtx, rty, rtz, rqx, rqy, rqz, rqw = rel
    t1x, t1y, t1z, ax, ay, az, aw = n1
    t2x, t2y, t2z, bx, by, bz, bw = n2c
    qbx = -rqw * ax + rqx * aw - rqy * az + rqz * ay
    qby = -rqw * ay + rqx * az + rqy * aw - rqz * ax
    qbz = -rqw * az - rqx * ay + rqy * ax + rqz * aw
    qbw = rqw * aw + rqx * ax + rqy * ay + rqz * az
    vx = t2x - t1x
    vy = t2y - t1y
    vz = t2z - t1z
    uvx = qby * vz - qbz * vy
    uvy = qbz * vx - qbx * vz
    uvz = qbx * vy - qby * vx
    tex = rtx + vx + 2.0 * (qbw * uvx + qby * uvz - qbz * uvy)
    tey = rty + vy + 2.0 * (qbw * uvy + qbz * uvx - qbx * uvz)
    tez = rtz + vz + 2.0 * (qbw * uvz + qbx * uvy - qby * uvx)
    qex = qbw * bx + qbx * bw + qby * bz - qbz * by
    qey = qbw * by - qbx * bz + qby * bw + qbz * bx
    qez = qbw * bz + qbx * by - qby * bx + qbz * bw
    qew = qbw * bw - qbx * bx - qby * by - qbz * bz
    return _se3_log_parts(tex, tey, tez, qex, qey, qez, qew)


def _sc_body(edges_hbm, rel_hbm, nodes_hbm, prior_out, err_out,
             edges_v, idx1_v, idx2_v, rel_v, n1_v, n2_v, out_v, p_row, p_out,
             sem):
    wid = lax.axis_index("s") * 2 + lax.axis_index("c")
    e_total = err_out.shape[0]
    per_w = e_total // 32
    n_chunks = per_w // _C
    lanes = lax.iota(jnp.int32, 16)
    zeros16 = jnp.zeros((16,), jnp.int32)
    ones16 = jnp.full((16,), 1, jnp.int32)

    def chunk_body(k, _):
        off = pl.multiple_of(wid * per_w + k * _C, _C)
        pltpu.sync_copy(edges_hbm.at[pl.ds(off, _C)], edges_v)
        pltpu.sync_copy(rel_hbm.at[pl.ds(off, _C)], rel_v)

        def deint_body(i, _):
            row = i * 16 + lanes
            idx1_v[pl.ds(i * 16, 16)] = plsc.load_gather(edges_v, [row, zeros16])
            idx2_v[pl.ds(i * 16, 16)] = plsc.load_gather(edges_v, [row, ones16])
            return ()

        lax.fori_loop(0, _C // 16, deint_body, (), unroll=False)

        descs = []
        for j in range(_IROWS):
            descs.append(pltpu.async_copy(
                nodes_hbm.at[idx1_v.at[pl.ds(j * _IW, _IW)]],
                n1_v.at[pl.ds(j * _IW, _IW)], sem))
            descs.append(pltpu.async_copy(
                nodes_hbm.at[idx2_v.at[pl.ds(j * _IW, _IW)]],
                n2_v.at[pl.ds(j * _IW, _IW)], sem))
        for d in descs:
            d.wait()

        def group_body(i, _):
            row = i * 16 + lanes

            def comp(ref, c):
                return plsc.load_gather(ref, [row, jnp.full((16,), c, jnp.int32)])

            rel = tuple(comp(rel_v, c) for c in range(7))
            g1 = tuple(comp(n1_v, c) for c in range(7))
            g2 = tuple(comp(n2_v, c) for c in range(7))
            res = _edge_error_log(rel, g1, g2)
            for c in range(6):
                plsc.store_scatter(out_v, [row, jnp.full((16,), c, jnp.int32)], res[c])
            return ()

        lax.fori_loop(0, _C // 16, group_body, (), unroll=False)
        pltpu.sync_copy(out_v, err_out.at[pl.ds(off, _C)])
        return ()

    lax.fori_loop(0, n_chunks, chunk_body, (), unroll=False)

    @pl.when(wid == 0)
    def _():
        pltpu.sync_copy(nodes_hbm.at[pl.ds(0, 1)], p_row)
        comps = tuple(
            plsc.load_gather(p_row, [zeros16, jnp.full((16,), c, jnp.int32)])
            for c in range(7))
        res = _se3_log_parts(*comps)
        mask0 = lanes == 0
        for c in range(6):
            plsc.store_scatter(p_out, [jnp.full((16,), c, jnp.int32)],
                               res[c], mask=mask0)
        pltpu.sync_copy(p_out.at[pl.ds(0, 6)], prior_out)


def kernel(edges, relative_poses, nodes):
    e_total = edges.shape[0]

    mesh = plsc.VectorSubcoreMesh(core_axis_name="c", subcore_axis_name="s")
    sc = pl.kernel(
        _sc_body,
        out_type=(
            jax.ShapeDtypeStruct((6,), jnp.float32),
            jax.ShapeDtypeStruct((e_total, 6), jnp.float32),
        ),
        mesh=mesh,
        compiler_params=pltpu.CompilerParams(
            needs_layout_passes=False, use_tc_tiling_on_sc=False),
        scratch_types=[
            pltpu.VMEM((_C, 2), jnp.int32),
            pltpu.VMEM((_C,), jnp.int32),
            pltpu.VMEM((_C,), jnp.int32),
            pltpu.VMEM((_C, 7), jnp.float32),
            pltpu.VMEM((_C, 7), jnp.float32),
            pltpu.VMEM((_C, 7), jnp.float32),
            pltpu.VMEM((_C, 6), jnp.float32),
            pltpu.VMEM((1, 7), jnp.float32),
            pltpu.VMEM((16,), jnp.float32),
            pltpu.SemaphoreType.DMA,
        ],
    )
    prior, err = sc(edges, relative_poses, nodes)
    return prior, err


# speedup vs baseline: 1.0008x; 1.0008x over previous
import functools

import jax
import jax.numpy as jnp
from jax import lax
from jax.experimental import pallas as pl
from jax.experimental.pallas import tpu as pltpu
from jax.experimental.pallas import tpu_sc as plsc

_EPS = 1e-6
_PI = 3.14159265358979323846
_HALF_PI = _PI / 2.0
_ATAN_C = (
    0.9999966347006731,
    -0.3331830289944677,
    0.19813213509068275,
    -0.1324752277162814,
    0.07981120495618609,
    -0.03372593810415406,
    0.006842624897572022,
)

_C = 1600
_IW = 80
_IROWS = _C // _IW


def _rsqrt(x):
    i = lax.bitcast_convert_type(x, jnp.int32)
    i = jnp.int32(0x5F3759DF) - lax.shift_right_arithmetic(i, 1)
    y = lax.bitcast_convert_type(i, jnp.float32)
    xh = x * 0.5
    y = y * (1.5 - xh * y * y)
    y = y * (1.5 - xh * y * y)
    y = y * (1.5 - xh * y * y)
    return y


def _atan2_pos(n, w):
    aw = jnp.abs(w)
    mn = jnp.minimum(n, aw)
    mx = jnp.maximum(n, aw)
    z = mn / jnp.maximum(mx, 1e-35)
    z2 = z * z
    p = jnp.float32(_ATAN_C[-1])
    for c in _ATAN_C[-2::-1]:
        p = p * z2 + jnp.float32(c)
    p = p * z
    r = jnp.where(n > aw, _HALF_PI - p, p)
    return jnp.where(w >= 0.0, r, _PI - r)


def _se3_log_parts(tx, ty, tz, qx, qy, qz, qw):
    n2 = qx * qx + qy * qy + qz * qz
    w2 = qw * qw
    s2 = n2 + w2
    n = n2 * _rsqrt(jnp.maximum(n2, 1e-35))
    theta = 2.0 * _atan2_pos(n, qw)
    n_safe = jnp.where(n > _EPS, n, 1.0)
    w_safe = jnp.where(jnp.abs(qw) > _EPS, qw, 1.0)
    scale = jnp.where(n > _EPS, theta / n_safe, 2.0 / w_safe)
    px = scale * qx
    py = scale * qy
    pz = scale * qz
    th = jnp.abs(scale) * n
    th_safe = jnp.where(th > _EPS, th, 1.0)
    inv_s2 = 1.0 / s2
    cth = (w2 - n2) * inv_s2
    sth = 2.0 * n * qw * inv_s2
    coef = jnp.where(
        th > _EPS,
        1.0 / (th_safe * th_safe)
        - (1.0 + cth) / (2.0 * th_safe * sth),
        1.0 / 12.0,
    )
    pvx = py * tz - pz * ty
    pvy = pz * tx - px * tz
    pvz = px * ty - py * tx
    ppvx = py * pvz - pz * pvy
    ppvy = pz * pvx - px * pvz
    ppvz = px * pvy - py * pvx
    rx = tx - 0.5 * pvx + coef * ppvx
    ry = ty - 0.5 * pvy + coef * ppvy
    rz = tz - 0.5 * pvz + coef * ppvz
    return rx, ry, rz, px, py, pz


def _edge_error_log(rel, n1, n2c):
    rtx, rty, rtz, rqx, rqy, rqz, rqw = rel
    t1x, t1y, t1z, ax, ay, az, aw = n1
    t2x, t2y, t2z, bx, by, bz, bw = n2c
    qbx = -rqw * ax + rqx * aw - rqy * az + rqz * ay
    qby = -rqw * ay + rqx * az + rqy * aw - rqz * ax
    qbz = -rqw * az - rqx * ay + rqy * ax + rqz * aw
    qbw = rqw * aw + rqx * ax + rqy * ay + rqz * az
    vx = t2x - t1x
    vy = t2y - t1y
    vz = t2z - t1z
    uvx = qby * vz - qbz * vy
    uvy = qbz * vx - qbx * vz
    uvz = qbx * vy - qby * vx
    tex = rtx + vx + 2.0 * (qbw * uvx + qby * uvz - qbz * uvy)
    tey = rty + vy + 2.0 * (qbw * uvy + qbz * uvx - qbx * uvz)
    tez = rtz + vz + 2.0 * (qbw * uvz + qbx * uvy - qby * uvx)
    qex = qbw * bx + qbx * bw + qby * bz - qbz * by
    qey = qbw * by - qbx * bz + qby * bw + qbz * bx
    qez = qbw * bz + qbx * by - qby * bx + qbz * bw
    qew = qbw * bw - qbx * bx - qby * by - qbz * bz
    return _se3_log_parts(tex, tey, tez, qex, qey, qez, qew)


def _sc_body(edges_hbm, rel_hbm, nodes_hbm, prior_out, err_out,
             edges_v, idx1_v, idx2_v, rel_v, n1_v, n2_v, out_v, p_row, p_out,
             sem):
    wid = lax.axis_index("s") * 2 + lax.axis_index("c")
    e_total = err_out.shape[0]
    per_w = e_total // 32
    n_chunks = per_w // _C
    lanes = lax.iota(jnp.int32, 16)
    zeros16 = jnp.zeros((16,), jnp.int32)
    ones16 = jnp.full((16,), 1, jnp.int32)

    def chunk_body(k, _):
        off = pl.multiple_of(wid * per_w + k * _C, _C)
        pltpu.sync_copy(edges_hbm.at[pl.ds(off, _C)], edges_v)
        pltpu.sync_copy(rel_hbm.at[pl.ds(off, _C)], rel_v)

        def deint_body(i, _):
            row = i * 16 + lanes
            idx1_v[pl.ds(i * 16, 16)] = plsc.load_gather(edges_v, [row, zeros16])
            idx2_v[pl.ds(i * 16, 16)] = plsc.load_gather(edges_v, [row, ones16])
            return ()

        lax.fori_loop(0, _C // 16, deint_body, (), unroll=False)

        descs = []
        for j in range(_IROWS):
            descs.append(pltpu.async_copy(
                nodes_hbm.at[idx1_v.at[pl.ds(j * _IW, _IW)]],
                n1_v.at[pl.ds(j * _IW, _IW)], sem))
            descs.append(pltpu.async_copy(
                nodes_hbm.at[idx2_v.at[pl.ds(j * _IW, _IW)]],
                n2_v.at[pl.ds(j * _IW, _IW)], sem))
        for d in descs:
            d.wait()

        def group_body(i, _):
            row = i * 16 + lanes

            def comp(ref, c):
                return plsc.load_gather(ref, [row, jnp.full((16,), c, jnp.int32)])

            rel = tuple(comp(rel_v, c) for c in range(7))
            g1 = tuple(comp(n1_v, c) for c in range(7))
            g2 = tuple(comp(n2_v, c) for c in range(7))
            res = _edge_error_log(rel, g1, g2)
            for c in range(6):
                plsc.store_scatter(out_v, [row, jnp.full((16,), c, jnp.int32)], res[c])
            return ()

        lax.fori_loop(0, _C // 16, group_body, (), unroll=False)
        pltpu.sync_copy(out_v, err_out.at[pl.ds(off, _C)])
        return ()

    lax.fori_loop(0, n_chunks, chunk_body, (), unroll=False)

    @pl.when(wid == 0)
    def _():
        pltpu.sync_copy(nodes_hbm.at[pl.ds(0, 1)], p_row)
        comps = tuple(
            plsc.load_gather(p_row, [zeros16, jnp.full((16,), c, jnp.int32)])
            for c in range(7))
        res = _se3_log_parts(*comps)
        mask0 = lanes == 0
        for c in range(6):
            plsc.store_scatter(p_out, [jnp.full((16,), c, jnp.int32)],
                               res[c], mask=mask0)
        pltpu.sync_copy(p_out.at[pl.ds(0, 6)], prior_out)


def kernel(edges, relative_poses, nodes):
    e_total = edges.shape[0]
    n_nodes = nodes.shape[0]
    nodes_pad = jnp.concatenate(
        [nodes, jnp.zeros((n_nodes, 1), jnp.float32)], axis=1)

    mesh = plsc.VectorSubcoreMesh(core_axis_name="c", subcore_axis_name="s")
    sc = pl.kernel(
        _sc_body,
        out_type=(
            jax.ShapeDtypeStruct((6,), jnp.float32),
            jax.ShapeDtypeStruct((e_total, 6), jnp.float32),
        ),
        mesh=mesh,
        compiler_params=pltpu.CompilerParams(
            needs_layout_passes=False, use_tc_tiling_on_sc=False),
        scratch_types=[
            pltpu.VMEM((_C, 2), jnp.int32),
            pltpu.VMEM((_C,), jnp.int32),
            pltpu.VMEM((_C,), jnp.int32),
            pltpu.VMEM((_C, 7), jnp.float32),
            pltpu.VMEM((_C, 8), jnp.float32),
            pltpu.VMEM((_C, 8), jnp.float32),
            pltpu.VMEM((_C, 6), jnp.float32),
            pltpu.VMEM((1, 8), jnp.float32),
            pltpu.VMEM((16,), jnp.float32),
            pltpu.SemaphoreType.DMA,
        ],
    )
    prior, err = sc(edges, relative_poses, nodes_pad)
    return prior, err


# speedup vs baseline: 6.0421x; 6.0376x over previous
import functools

import jax
import jax.numpy as jnp
from jax import lax
from jax.experimental import pallas as pl
from jax.experimental.pallas import tpu as pltpu
from jax.experimental.pallas import tpu_sc as plsc

_EPS = 1e-6
_PI = 3.14159265358979323846
_HALF_PI = _PI / 2.0
_ATAN_C = (
    0.9999966347006731,
    -0.3331830289944677,
    0.19813213509068275,
    -0.1324752277162814,
    0.07981120495618609,
    -0.03372593810415406,
    0.006842624897572022,
)

_NW = 32
_CB = 12
_CA = _CB * 128
_C = 1600
_IW = 80


def _rsqrt(x):
    i = lax.bitcast_convert_type(x, jnp.int32)
    i = jnp.int32(0x5F3759DF) - lax.shift_right_arithmetic(i, 1)
    y = lax.bitcast_convert_type(i, jnp.float32)
    xh = x * 0.5
    y = y * (1.5 - xh * y * y)
    y = y * (1.5 - xh * y * y)
    return y


def _atan2_pos(n, w):
    aw = jnp.abs(w)
    mn = jnp.minimum(n, aw)
    mx = jnp.maximum(n, aw)
    z = mn / jnp.maximum(mx, 1e-35)
    z2 = z * z
    p = jnp.float32(_ATAN_C[-1])
    for c in _ATAN_C[-2::-1]:
        p = p * z2 + jnp.float32(c)
    p = p * z
    r = jnp.where(n > aw, _HALF_PI - p, p)
    return jnp.where(w >= 0.0, r, _PI - r)


def _se3_log_parts(tx, ty, tz, qx, qy, qz, qw):
    n2 = qx * qx + qy * qy + qz * qz
    w2 = qw * qw
    s2 = n2 + w2
    n = n2 * _rsqrt(jnp.maximum(n2, 1e-35))
    theta = 2.0 * _atan2_pos(n, qw)
    n_safe = jnp.where(n > _EPS, n, 1.0)
    w_safe = jnp.where(jnp.abs(qw) > _EPS, qw, 1.0)
    scale = jnp.where(n > _EPS, theta / n_safe, 2.0 / w_safe)
    px = scale * qx
    py = scale * qy
    pz = scale * qz
    th = jnp.abs(scale) * n
    th_safe = jnp.where(th > _EPS, th, 1.0)
    inv_s2 = 1.0 / s2
    cth = (w2 - n2) * inv_s2
    sth = 2.0 * n * qw * inv_s2
    coef = jnp.where(
        th > _EPS,
        1.0 / (th_safe * th_safe)
        - (1.0 + cth) / (2.0 * th_safe * sth),
        1.0 / 12.0,
    )
    pvx = py * tz - pz * ty
    pvy = pz * tx - px * tz
    pvz = px * ty - py * tx
    ppvx = py * pvz - pz * pvy
    ppvy = pz * pvx - px * pvz
    ppvz = px * pvy - py * pvx
    rx = tx - 0.5 * pvx + coef * ppvx
    ry = ty - 0.5 * pvy + coef * ppvy
    rz = tz - 0.5 * pvz + coef * ppvz
    return rx, ry, rz, px, py, pz


def _edge_error_log(rel, n1, n2c):
    rtx, rty, rtz, rqx, rqy, rqz, rqw = rel
    t1x, t1y, t1z, ax, ay, az, aw = n1
    t2x, t2y, t2z, bx, by, bz, bw = n2c
    qbx = -rqw * ax + rqx * aw - rqy * az + rqz * ay
    qby = -rqw * ay + rqx * az + rqy * aw - rqz * ax
    qbz = -rqw * az - rqx * ay + rqy * ax + rqz * aw
    qbw = rqw * aw + rqx * ax + rqy * ay + rqz * az
    vx = t2x - t1x
    vy = t2y - t1y
    vz = t2z - t1z
    uvx = qby * vz - qbz * vy
    uvy = qbz * vx - qbx * vz
    uvz = qbx * vy - qby * vx
    tex = rtx + vx + 2.0 * (qbw * uvx + qby * uvz - qbz * uvy)
    tey = rty + vy + 2.0 * (qbw * uvy + qbz * uvx - qbx * uvz)
    tez = rtz + vz + 2.0 * (qbw * uvz + qbx * uvy - qby * uvx)
    qex = qbw * bx + qbx * bw + qby * bz - qbz * by
    qey = qbw * by - qbx * bz + qby * bw + qbz * bx
    qez = qbw * bz + qbx * by - qby * bx + qbz * bw
    qew = qbw * bw - qbx * bx - qby * by - qbz * bz
    return _se3_log_parts(tex, tey, tez, qex, qey, qez, qew)


def _stage_a_body(edges_hbm, rel_hbm, nodes_hbm,
                  i1_out, i2_out, rel_out, tab_out,
                  ei_v, rv_v, i1b, i2b, rlb, nb_v, aos_v, sem_i, sem_o):
    sid = lax.axis_index("s")
    wid = sid * 2 + lax.axis_index("c")
    e_total = i1_out.shape[0]
    n_blocks = e_total // 128
    n_chunks = n_blocks // _CB
    tail_blocks = n_blocks - n_chunks * _CB
    n_pad_nodes = nodes_hbm.shape[1]
    lanes = lax.iota(jnp.int32, 16)
    zeros16f = jnp.zeros((16,), jnp.float32)

    n_units = n_pad_nodes // 128
    my_units = (n_units - wid + _NW - 1) // _NW

    def build_body(i, _):
        u = wid + i * _NW
        col0 = pl.multiple_of(u * 128, 128)
        pltpu.sync_copy(nodes_hbm.at[:, pl.ds(col0, 128)], nb_v)

        def g_body(g, _):
            flat = (g * 16 + lanes) * 8
            for c in range(7):
                v = nb_v[c, pl.ds(g * 16, 16)]
                plsc.store_scatter(aos_v, [flat + c], v)
            plsc.store_scatter(aos_v, [flat + 7], zeros16f)
            return ()

        lax.fori_loop(0, 8, g_body, (), unroll=True)
        pltpu.sync_copy(aos_v, tab_out.at[pl.ds(u * 1024, 1024)])
        return ()

    lax.fori_loop(0, my_units, build_body, (), unroll=False)

    def drain_outs(buf):
        pltpu.make_async_copy(i1_out.at[pl.ds(0, _CA)], i1b.at[buf],
                              sem_o).wait()
        pltpu.make_async_copy(i1_out.at[pl.ds(0, _CA)], i2b.at[buf],
                              sem_o).wait()
        pltpu.make_async_copy(rel_out.at[pl.ds(0, 7 * _CA)], rlb.at[buf],
                              sem_o).wait()

    def chunk_work(k, col0):
        buf = jnp.bitwise_and(k, 1)
        d_e = pltpu.async_copy(edges_hbm.at[:, pl.ds(col0, _CA)], ei_v, sem_i)
        d_r = pltpu.async_copy(rel_hbm.at[:, pl.ds(col0, _CA)], rv_v, sem_i)
        d_e.wait()
        d_r.wait()

        @pl.when(k > 0)
        def _():
            drain_outs(1 - buf)

        def copy_body(g, _):
            s = pl.ds(g * 16, 16)
            i1b[buf, s] = ei_v[0, s]
            i2b[buf, s] = ei_v[1, s]
            for cc in range(7):
                rlb[buf, pl.ds(cc * _CA + g * 16, 16)] = rv_v[cc, s]
            return ()

        lax.fori_loop(0, _CA // 16, copy_body, (), unroll=4)
        pltpu.async_copy(i1b.at[buf], i1_out.at[pl.ds(col0, _CA)], sem_o)
        pltpu.async_copy(i2b.at[buf], i2_out.at[pl.ds(col0, _CA)], sem_o)
        for cc in range(7):
            pltpu.async_copy(rlb.at[buf, pl.ds(cc * _CA, _CA)],
                             rel_out.at[pl.ds(cc * e_total + col0, _CA)],
                             sem_o)

    def chunk_body(i, _):
        cid = wid + i * _NW
        chunk_work(i, pl.multiple_of(cid * _CA, 128))
        return ()

    my_chunks = (n_chunks - wid + _NW - 1) // _NW
    lax.fori_loop(0, my_chunks, chunk_body, (), unroll=False)

    @pl.when(my_chunks > 0)
    def _():
        drain_outs(jnp.bitwise_and(my_chunks - 1, 1))

    @pl.when(wid < tail_blocks)
    def _():
        col0 = pl.multiple_of(n_chunks * _CA + wid * 128, 128)
        c = 128
        pltpu.sync_copy(edges_hbm.at[:, pl.ds(col0, c)],
                        ei_v.at[:, pl.ds(0, c)])
        pltpu.sync_copy(rel_hbm.at[:, pl.ds(col0, c)],
                        rv_v.at[:, pl.ds(0, c)])

        def copy_body(g, _):
            s = pl.ds(g * 16, 16)
            i1b[0, s] = ei_v[0, s]
            i2b[0, s] = ei_v[1, s]
            for cc in range(7):
                rlb[0, pl.ds(cc * _CA + g * 16, 16)] = rv_v[cc, s]
            return ()

        lax.fori_loop(0, c // 16, copy_body, (), unroll=False)
        pltpu.sync_copy(i1b.at[0, pl.ds(0, c)], i1_out.at[pl.ds(col0, c)])
        pltpu.sync_copy(i2b.at[0, pl.ds(0, c)], i2_out.at[pl.ds(col0, c)])
        for cc in range(7):
            pltpu.sync_copy(rlb.at[0, pl.ds(cc * _CA, c)],
                            rel_out.at[pl.ds(cc * e_total + col0, c)])


def _stage_b_body(i1_hbm, i2_hbm, rel_hbm, tab_hbm, prior_out, err_out,
                  i1_v, i2_v, rel_v, n1_v, n2_v, out_v, p_row, p_out,
                  sem_i, sem_r, sem_g, sem_o):
    wid = lax.axis_index("s") * 2 + lax.axis_index("c")
    e_total = i1_hbm.shape[0]
    per_w = e_total // _NW
    n_chunks = per_w // _C
    lanes = lax.iota(jnp.int32, 16)
    zeros16 = jnp.zeros((16,), jnp.int32)

    def chunk_body(k, _):
        off = pl.multiple_of(wid * per_w + k * _C, _C)
        d_i1 = pltpu.async_copy(i1_hbm.at[pl.ds(off, _C)], i1_v, sem_i)
        d_i2 = pltpu.async_copy(i2_hbm.at[pl.ds(off, _C)], i2_v, sem_i)
        rdescs = [
            pltpu.async_copy(rel_hbm.at[pl.ds(c * e_total + off, _C)],
                             rel_v.at[pl.ds(c * _C, _C)], sem_r)
            for c in range(7)
        ]
        d_i1.wait()
        d_i2.wait()
        descs = []
        for j in range(_C // _IW):
            descs.append(pltpu.async_copy(
                tab_hbm.at[i1_v.at[pl.ds(j * _IW, _IW)]],
                n1_v.at[pl.ds(j * _IW, _IW)], sem_g))
            descs.append(pltpu.async_copy(
                tab_hbm.at[i2_v.at[pl.ds(j * _IW, _IW)]],
                n2_v.at[pl.ds(j * _IW, _IW)], sem_g))
        for d in rdescs:
            d.wait()
        for d in descs:
            d.wait()
        buf = jnp.bitwise_and(k, 1)
        @pl.when(k > 0)
        def _():
            pltpu.make_async_copy(
                err_out.at[pl.ds(0, 6 * _C)],
                out_v.at[1 - buf], sem_o).wait()

        def group_body(i, _):
            row = i * 16 + lanes

            def gcomp(ref, c_):
                return plsc.load_gather(ref, [row, jnp.full((16,), c_, jnp.int32)])

            rel = tuple(rel_v[pl.ds(c_ * _C + i * 16, 16)] for c_ in range(7))
            g1 = tuple(gcomp(n1_v, c_) for c_ in range(7))
            g2 = tuple(gcomp(n2_v, c_) for c_ in range(7))
            res = _edge_error_log(rel, g1, g2)
            for c_ in range(6):
                out_v[buf, pl.ds(c_ * _C + i * 16, 16)] = res[c_]
            return ()

        lax.fori_loop(0, _C // 16, group_body, (), unroll=2)
        for c in range(6):
            pltpu.async_copy(out_v.at[buf, pl.ds(c * _C, _C)],
                             err_out.at[pl.ds(c * e_total + off, _C)], sem_o)
        return ()

    lax.fori_loop(0, n_chunks, chunk_body, (), unroll=False)
    pltpu.make_async_copy(
        err_out.at[pl.ds(0, 6 * _C)],
        out_v.at[jnp.bitwise_and(n_chunks - 1, 1)], sem_o).wait()

    @pl.when(wid == 0)
    def _():
        pltpu.sync_copy(tab_hbm.at[pl.ds(0, 1)], p_row)
        comps = tuple(
            plsc.load_gather(p_row, [zeros16, jnp.full((16,), c, jnp.int32)])
            for c in range(7))
        res = _se3_log_parts(*comps)
        mask0 = lanes == 0
        for c in range(6):
            plsc.store_scatter(p_out, [jnp.full((16,), c, jnp.int32)],
                               res[c], mask=mask0)
        pltpu.sync_copy(p_out.at[pl.ds(0, 6)], prior_out)


def _stage_c_body(err_lin_hbm, err_out, cv, ov):
    wid = lax.axis_index("s") * 2 + lax.axis_index("c")
    e_total = err_out.shape[1]
    n_blocks = e_total // 128
    n_chunks = n_blocks // _CB
    tail_blocks = n_blocks - n_chunks * _CB

    def chunk_work(col0, cb):
        c = cb * 128
        for cc in range(6):
            pltpu.sync_copy(err_lin_hbm.at[pl.ds(cc * e_total + col0, c)],
                            cv.at[pl.ds(cc * _CA, c)])

        def copy_body(g, _):
            for cc in range(6):
                ov[cc, pl.ds(g * 16, 16)] = cv[pl.ds(cc * _CA + g * 16, 16)]
            return ()

        lax.fori_loop(0, c // 16, copy_body, (), unroll=4)
        pltpu.sync_copy(ov.at[:, pl.ds(0, c)], err_out.at[:, pl.ds(col0, c)])

    def chunk_body(i, _):
        cid = wid + i * _NW
        chunk_work(pl.multiple_of(cid * _CA, 128), _CB)
        return ()

    my_chunks = (n_chunks - wid + _NW - 1) // _NW
    lax.fori_loop(0, my_chunks, chunk_body, (), unroll=False)

    @pl.when(wid < tail_blocks)
    def _():
        chunk_work(pl.multiple_of(n_chunks * _CA + wid * 128, 128), 1)


def kernel(edges, relative_poses, nodes):
    e_total = edges.shape[0]
    n_nodes = nodes.shape[0]
    n_pad = (-n_nodes) % 128
    npad = n_nodes + n_pad
    nodes_t = jnp.pad(nodes.T, ((0, 0), (0, n_pad)))

    mesh = plsc.VectorSubcoreMesh(core_axis_name="c", subcore_axis_name="s")

    stage_a = pl.kernel(
        _stage_a_body,
        out_type=(
            jax.ShapeDtypeStruct((e_total,), jnp.int32),
            jax.ShapeDtypeStruct((e_total,), jnp.int32),
            jax.ShapeDtypeStruct((7 * e_total,), jnp.float32),
            jax.ShapeDtypeStruct((npad * 8,), jnp.float32),
        ),
        mesh=mesh,
        compiler_params=pltpu.CompilerParams(needs_layout_passes=False),
        scratch_types=[
            pltpu.VMEM((2, _CA), jnp.int32),
            pltpu.VMEM((7, _CA), jnp.float32),
            pltpu.VMEM((2, _CA), jnp.int32),
            pltpu.VMEM((2, _CA), jnp.int32),
            pltpu.VMEM((2, 7 * _CA), jnp.float32),
            pltpu.VMEM((7, 128), jnp.float32),
            pltpu.VMEM((1024,), jnp.float32),
            pltpu.SemaphoreType.DMA,
            pltpu.SemaphoreType.DMA,
        ],
    )
    idx1, idx2, rel_lin, tab_lin = stage_a(
        edges.T, relative_poses.T, nodes_t)

    stage_b = pl.kernel(
        _stage_b_body,
        out_type=(
            jax.ShapeDtypeStruct((6,), jnp.float32),
            jax.ShapeDtypeStruct((6 * e_total,), jnp.float32),
        ),
        mesh=mesh,
        compiler_params=pltpu.CompilerParams(
            needs_layout_passes=False, use_tc_tiling_on_sc=False),
        scratch_types=[
            pltpu.VMEM((_C,), jnp.int32),
            pltpu.VMEM((_C,), jnp.int32),
            pltpu.VMEM((7 * _C,), jnp.float32),
            pltpu.VMEM((_C, 8), jnp.float32),
            pltpu.VMEM((_C, 8), jnp.float32),
            pltpu.VMEM((2, 6 * _C), jnp.float32),
            pltpu.VMEM((1, 8), jnp.float32),
            pltpu.VMEM((16,), jnp.float32),
            pltpu.SemaphoreType.DMA,
            pltpu.SemaphoreType.DMA,
            pltpu.SemaphoreType.DMA,
            pltpu.SemaphoreType.DMA,
        ],
    )
    prior, err_lin = stage_b(idx1, idx2, rel_lin, tab_lin.reshape(npad, 8))

    stage_c = pl.kernel(
        _stage_c_body,
        out_type=jax.ShapeDtypeStruct((6, e_total), jnp.float32),
        mesh=mesh,
        compiler_params=pltpu.CompilerParams(needs_layout_passes=False),
        scratch_types=[
            pltpu.VMEM((6 * _CA,), jnp.float32),
            pltpu.VMEM((6, _CA), jnp.float32),
        ],
    )
    err_t = stage_c(err_lin)
    return prior, err_t.T
